# Initial kernel scaffold; baseline (speedup 1.0000x reference)
import functools

import jax
import jax.numpy as jnp
from jax import lax
from jax.experimental import pallas as pl
from jax.experimental.pallas import tpu as pltpu
from jax.experimental.pallas import tpu_sc as plsc

N = 50000
D_IN = 128
DH = 64
DHALF = 32
E = 800000

NUM_SUBCORES = 16
SUB = 128
NSUB = 8
CHUNK = SUB * NSUB
NCHUNKS = 50
EPT = CHUNK * NCHUNKS
E_PAD = EPT * NUM_SUBCORES
IDX_ROWS_PER_TILE = EPT // SUB

ACC_ROWS = N + 8
ZCH = 625
ZPT = N // NUM_SUBCORES
NZ = ZPT // ZCH

BS = 1000


def _proj_ln_body(x_ref, w_ref, b_ref, g_ref, b2_ref, lo_ref, hi_ref):
    y = jnp.dot(x_ref[...], w_ref[...], preferred_element_type=jnp.float32)
    y = y + b_ref[...]
    m = jnp.mean(y, axis=-1, keepdims=True)
    v = jnp.mean((y - m) ** 2, axis=-1, keepdims=True)
    y = (y - m) * lax.rsqrt(v + 1e-5) * g_ref[...] + b2_ref[...]
    lo_ref[...] = y[:, :DHALF]
    hi_ref[...] = y[:, DHALF:]


def _proj_ln(x, w, b, g, b2):
    return pl.pallas_call(
        _proj_ln_body,
        grid=(N // BS,),
        in_specs=[
            pl.BlockSpec((BS, D_IN), lambda i: (i, 0)),
            pl.BlockSpec((D_IN, DH), lambda i: (0, 0)),
            pl.BlockSpec((1, DH), lambda i: (0, 0)),
            pl.BlockSpec((1, DH), lambda i: (0, 0)),
            pl.BlockSpec((1, DH), lambda i: (0, 0)),
        ],
        out_specs=[
            pl.BlockSpec((BS, DHALF), lambda i: (i, 0)),
            pl.BlockSpec((BS, DHALF), lambda i: (i, 0)),
        ],
        out_shape=[
            jax.ShapeDtypeStruct((N, DHALF), jnp.float32),
            jax.ShapeDtypeStruct((N, DHALF), jnp.float32),
        ],
    )(x, w, b.reshape(1, DH), g.reshape(1, DH), b2.reshape(1, DH))


def _sage_dense_body(relu, split, sl_ref, sh_ref, c_ref, hl_ref, hh_ref,
                     wl_ref, wr_ref, b_ref, *out_refs):
    cnt = c_ref[...][:, 0:1]
    r = 1.0 / jnp.maximum(cnt, 1.0)
    ml = sl_ref[...] * r
    mh = sh_ref[...] * r
    y = jnp.dot(ml, wl_ref[:DHALF, :], preferred_element_type=jnp.float32)
    y = y + jnp.dot(mh, wl_ref[DHALF:, :], preferred_element_type=jnp.float32)
    y = y + jnp.dot(hl_ref[...], wr_ref[:DHALF, :],
                    preferred_element_type=jnp.float32)
    y = y + jnp.dot(hh_ref[...], wr_ref[DHALF:, :],
                    preferred_element_type=jnp.float32)
    y = y + b_ref[...]
    if relu:
        y = jnp.maximum(y, 0.0)
    if split:
        out_refs[0][...] = y[:, :DHALF]
        out_refs[1][...] = y[:, DHALF:]
    else:
        out_refs[0][...] = y


def _sage_dense(sl, sh, cnt, hl, hh, wl, wr, b, relu, split):
    if split:
        out_specs = [pl.BlockSpec((BS, DHALF), lambda i: (i, 0)),
                     pl.BlockSpec((BS, DHALF), lambda i: (i, 0))]
        out_shape = [jax.ShapeDtypeStruct((N, DHALF), jnp.float32),
                     jax.ShapeDtypeStruct((N, DHALF), jnp.float32)]
    else:
        out_specs = [pl.BlockSpec((BS, DH), lambda i: (i, 0))]
        out_shape = [jax.ShapeDtypeStruct((N, DH), jnp.float32)]
    return pl.pallas_call(
        functools.partial(_sage_dense_body, relu, split),
        grid=(N // BS,),
        in_specs=[
            pl.BlockSpec((BS, DHALF), lambda i: (i, 0)),
            pl.BlockSpec((BS, DHALF), lambda i: (i, 0)),
            pl.BlockSpec((BS, 16), lambda i: (i, 0)),
            pl.BlockSpec((BS, DHALF), lambda i: (i, 0)),
            pl.BlockSpec((BS, DHALF), lambda i: (i, 0)),
            pl.BlockSpec((DH, DH), lambda i: (0, 0)),
            pl.BlockSpec((DH, DH), lambda i: (0, 0)),
            pl.BlockSpec((1, DH), lambda i: (0, 0)),
        ],
        out_specs=out_specs,
        out_shape=out_shape,
    )(sl, sh, cnt, hl, hh, wl, wr, b.reshape(1, DH))


def _segsum_kernel_body(hlo, hhi, src_h, dst_h, zeros_h, out_lo, out_hi,
                        sidx, didx, rows, zbuf, acc, sem):
    c = lax.axis_index("c")
    s = lax.axis_index("s")

    def run(table, out):
        zbase = s * ZPT
        pltpu.sync_copy(zeros_h, zbuf)
        for i in range(NZ):
            pltpu.sync_copy(zbuf, acc.at[pl.ds(zbase + i * ZCH, ZCH)])
        plsc.subcore_barrier()

        rbase = s * IDX_ROWS_PER_TILE

        def chunk(ci, carry):
            r0 = rbase + ci * NSUB
            pltpu.sync_copy(src_h.at[pl.ds(r0, NSUB)], sidx)
            pltpu.sync_copy(dst_h.at[pl.ds(r0, NSUB)], didx)
            handles = []
            for j in range(NSUB):
                handles.append(pltpu.async_copy(
                    table.at[sidx.at[j]],
                    rows.at[pl.ds(j * SUB, SUB)], sem))
            for h in handles:
                h.wait()
            for j in range(NSUB):
                pltpu.sync_copy(rows.at[pl.ds(j * SUB, SUB)],
                                acc.at[didx.at[j]], add=True)
            return carry

        lax.fori_loop(0, NCHUNKS, chunk, 0)
        plsc.subcore_barrier()
        for i in range(NZ):
            r0 = zbase + i * ZCH
            pltpu.sync_copy(acc.at[pl.ds(r0, ZCH)], zbuf)
            pltpu.sync_copy(zbuf, out.at[pl.ds(r0, ZCH)])

    @pl.when(c == 0)
    def _():
        run(hlo, out_lo)

    @pl.when(c == 1)
    def _():
        run(hhi, out_hi)


def _segsum(hlo, hhi, src2d, dst2d):
    zeros = jnp.zeros((ZCH, DHALF), jnp.float32)
    mesh = plsc.VectorSubcoreMesh(core_axis_name="c", subcore_axis_name="s")
    f = pl.kernel(
        _segsum_kernel_body,
        out_type=[jax.ShapeDtypeStruct((N, DHALF), jnp.float32),
                  jax.ShapeDtypeStruct((N, DHALF), jnp.float32)],
        mesh=mesh,
        scratch_types=[
            pltpu.VMEM((NSUB, SUB), jnp.int32),
            pltpu.VMEM((NSUB, SUB), jnp.int32),
            pltpu.VMEM((CHUNK, DHALF), jnp.float32),
            pltpu.VMEM((ZCH, DHALF), jnp.float32),
            pltpu.VMEM_SHARED((ACC_ROWS, DHALF), jnp.float32),
            pltpu.SemaphoreType.DMA,
        ],
    )
    return f(hlo, hhi, src2d, dst2d, zeros)


def _counts_kernel_body(dstA_h, dstB_h, zeros_h, ones_h, outA, outB,
                        didx, ones_v, zbuf, acc):
    c = lax.axis_index("c")
    s = lax.axis_index("s")

    def run(dst_h, out):
        zbase = s * ZPT
        pltpu.sync_copy(zeros_h, zbuf)
        for i in range(NZ):
            pltpu.sync_copy(zbuf, acc.at[pl.ds(zbase + i * ZCH, ZCH)])
        pltpu.sync_copy(ones_h, ones_v)
        plsc.subcore_barrier()

        rbase = s * IDX_ROWS_PER_TILE

        def chunk(ci, carry):
            r0 = rbase + ci * NSUB
            pltpu.sync_copy(dst_h.at[pl.ds(r0, NSUB)], didx)
            for j in range(NSUB):
                pltpu.sync_copy(ones_v, acc.at[didx.at[j]], add=True)
            return carry

        lax.fori_loop(0, NCHUNKS, chunk, 0)
        plsc.subcore_barrier()
        for i in range(NZ):
            r0 = zbase + i * ZCH
            pltpu.sync_copy(acc.at[pl.ds(r0, ZCH)], zbuf)
            pltpu.sync_copy(zbuf, out.at[pl.ds(r0, ZCH)])

    @pl.when(c == 0)
    def _():
        run(dstA_h, outA)

    @pl.when(c == 1)
    def _():
        run(dstB_h, outB)


def _counts(dstA2d, dstB2d):
    zeros = jnp.zeros((ZCH, 16), jnp.float32)
    ones = jnp.ones((SUB, 16), jnp.float32)
    mesh = plsc.VectorSubcoreMesh(core_axis_name="c", subcore_axis_name="s")
    f = pl.kernel(
        _counts_kernel_body,
        out_type=[jax.ShapeDtypeStruct((N, 16), jnp.float32),
                  jax.ShapeDtypeStruct((N, 16), jnp.float32)],
        mesh=mesh,
        scratch_types=[
            pltpu.VMEM((NSUB, SUB), jnp.int32),
            pltpu.VMEM((SUB, 16), jnp.float32),
            pltpu.VMEM((ZCH, 16), jnp.float32),
            pltpu.VMEM_SHARED((ACC_ROWS, 16), jnp.float32),
        ],
    )
    return f(dstA2d, dstB2d, zeros, ones)


def _pad_edges(ei):
    src = ei[0].astype(jnp.int32)
    dst = ei[1].astype(jnp.int32)
    npad = E_PAD - E
    src_p = jnp.concatenate([src, jnp.zeros((npad,), jnp.int32)])
    dst_p = jnp.concatenate([dst, jnp.full((npad,), N, jnp.int32)])
    return src_p.reshape(-1, SUB), dst_p.reshape(-1, SUB)


def kernel(x_user, x_item, edge_index_user_to_item, edge_index_item_rev_user,
           Wp_user, bp_user, Wp_item, bp_item, ln_g, ln_b,
           Wl_u2i_0, Wr_u2i_0, b_u2i_0, Wl_i2u_0, Wr_i2u_0, b_i2u_0,
           Wl_u2i_1, Wr_u2i_1, b_u2i_1, Wl_i2u_1, Wr_i2u_1, b_i2u_1):
    src_u2i, dst_u2i = _pad_edges(edge_index_user_to_item)
    src_i2u, dst_i2u = _pad_edges(edge_index_item_rev_user)

    hu_lo, hu_hi = _proj_ln(x_user, Wp_user, bp_user, ln_g, ln_b)
    hi_lo, hi_hi = _proj_ln(x_item, Wp_item, bp_item, ln_g, ln_b)

    cnt_i, cnt_u = _counts(dst_u2i, dst_i2u)

    si_lo, si_hi = _segsum(hu_lo, hu_hi, src_u2i, dst_u2i)
    su_lo, su_hi = _segsum(hi_lo, hi_hi, src_i2u, dst_i2u)

    hi1_lo, hi1_hi = _sage_dense(si_lo, si_hi, cnt_i, hi_lo, hi_hi,
                                 Wl_u2i_0, Wr_u2i_0, b_u2i_0,
                                 relu=True, split=True)
    hu1_lo, hu1_hi = _sage_dense(su_lo, su_hi, cnt_u, hu_lo, hu_hi,
                                 Wl_i2u_0, Wr_i2u_0, b_i2u_0,
                                 relu=True, split=True)

    si1_lo, si1_hi = _segsum(hu1_lo, hu1_hi, src_u2i, dst_u2i)
    su1_lo, su1_hi = _segsum(hi1_lo, hi1_hi, src_i2u, dst_i2u)

    (hi2,) = _sage_dense(si1_lo, si1_hi, cnt_i, hi1_lo, hi1_hi,
                         Wl_u2i_1, Wr_u2i_1, b_u2i_1,
                         relu=False, split=False)
    (hu2,) = _sage_dense(su1_lo, su1_hi, cnt_u, hu1_lo, hu1_hi,
                         Wl_i2u_1, Wr_i2u_1, b_i2u_1,
                         relu=False, split=False)

    return jnp.stack([hu2, hi2], axis=0)


# speedup vs baseline: 6.4918x; 6.4918x over previous
import functools

import jax
import jax.numpy as jnp
from jax import lax
from jax.experimental import pallas as pl
from jax.experimental.pallas import tpu as pltpu
from jax.experimental.pallas import tpu_sc as plsc

N = 50000
D_IN = 128
DH = 64
DHALF = 32
E = 800000

NUM_SUBCORES = 16
SUB = 128
NRING = 6
BODY_IDX_ROWS = 8
NBODIES = 50
EPT = BODY_IDX_ROWS * SUB * NBODIES
E_PAD = EPT * NUM_SUBCORES
NBODY_TOTAL = E_PAD // (BODY_IDX_ROWS * SUB)

N_PAD = 50048
ACC_ROWS = N_PAD
ZPT = N_PAD // NUM_SUBCORES
ZCHUNKS = ((0, 768), (768, 768), (1536, 768), (2304, 768), (3072, 56))
ZBUF = 768

CW = 8
BS = 1000


def _proj_ln_body(x_ref, w_ref, b_ref, g_ref, b2_ref, lo_ref, hi_ref):
    y = jnp.dot(x_ref[...], w_ref[...], preferred_element_type=jnp.float32)
    y = y + b_ref[...]
    m = jnp.mean(y, axis=-1, keepdims=True)
    v = jnp.mean((y - m) ** 2, axis=-1, keepdims=True)
    y = (y - m) * lax.rsqrt(v + 1e-5) * g_ref[...] + b2_ref[...]
    lo_ref[...] = y[:, :DHALF]
    hi_ref[...] = y[:, DHALF:]


def _proj_ln(x, w, b, g, b2):
    return pl.pallas_call(
        _proj_ln_body,
        grid=(N // BS,),
        in_specs=[
            pl.BlockSpec((BS, D_IN), lambda i: (i, 0)),
            pl.BlockSpec((D_IN, DH), lambda i: (0, 0)),
            pl.BlockSpec((1, DH), lambda i: (0, 0)),
            pl.BlockSpec((1, DH), lambda i: (0, 0)),
            pl.BlockSpec((1, DH), lambda i: (0, 0)),
        ],
        out_specs=[
            pl.BlockSpec((BS, DHALF), lambda i: (i, 0)),
            pl.BlockSpec((BS, DHALF), lambda i: (i, 0)),
        ],
        out_shape=[
            jax.ShapeDtypeStruct((N, DHALF), jnp.float32),
            jax.ShapeDtypeStruct((N, DHALF), jnp.float32),
        ],
    )(x, w, b.reshape(1, DH), g.reshape(1, DH), b2.reshape(1, DH))


def _dense_compute(sl_ref, sh_ref, c_ref, hl_ref, hh_ref, wl_ref, wr_ref,
                   b_ref):
    cnt = c_ref[...][:, 0:1]
    r = 1.0 / jnp.maximum(cnt, 1.0)
    ml = sl_ref[...] * r
    mh = sh_ref[...] * r
    y = jnp.dot(ml, wl_ref[:DHALF, :], preferred_element_type=jnp.float32)
    y = y + jnp.dot(mh, wl_ref[DHALF:, :], preferred_element_type=jnp.float32)
    y = y + jnp.dot(hl_ref[...], wr_ref[:DHALF, :],
                    preferred_element_type=jnp.float32)
    y = y + jnp.dot(hh_ref[...], wr_ref[DHALF:, :],
                    preferred_element_type=jnp.float32)
    return y + b_ref[...]


def _sage_mid_body(sl_ref, sh_ref, c_ref, hl_ref, hh_ref, wl_ref, wr_ref,
                   b_ref, lo_ref, hi_ref):
    y = _dense_compute(sl_ref, sh_ref, c_ref, hl_ref, hh_ref, wl_ref, wr_ref,
                       b_ref)
    y = jnp.maximum(y, 0.0)
    lo_ref[...] = y[:, :DHALF]
    hi_ref[...] = y[:, DHALF:]


_DENSE_IN_SPECS = [
    pl.BlockSpec((BS, DHALF), lambda i: (i, 0)),
    pl.BlockSpec((BS, DHALF), lambda i: (i, 0)),
    pl.BlockSpec((BS, CW), lambda i: (i, 0)),
    pl.BlockSpec((BS, DHALF), lambda i: (i, 0)),
    pl.BlockSpec((BS, DHALF), lambda i: (i, 0)),
    pl.BlockSpec((DH, DH), lambda i: (0, 0)),
    pl.BlockSpec((DH, DH), lambda i: (0, 0)),
    pl.BlockSpec((1, DH), lambda i: (0, 0)),
]


def _sage_mid(sl, sh, cnt, hl, hh, wl, wr, b):
    return pl.pallas_call(
        _sage_mid_body,
        grid=(N // BS,),
        in_specs=_DENSE_IN_SPECS,
        out_specs=[pl.BlockSpec((BS, DHALF), lambda i: (i, 0)),
                   pl.BlockSpec((BS, DHALF), lambda i: (i, 0))],
        out_shape=[jax.ShapeDtypeStruct((N, DHALF), jnp.float32),
                   jax.ShapeDtypeStruct((N, DHALF), jnp.float32)],
    )(sl, sh, cnt, hl, hh, wl, wr, b.reshape(1, DH))


def _sage_final_body(slot, sl_ref, sh_ref, c_ref, hl_ref, hh_ref, wl_ref,
                     wr_ref, b_ref, *refs):
    out_ref = refs[-1]
    y = _dense_compute(sl_ref, sh_ref, c_ref, hl_ref, hh_ref, wl_ref, wr_ref,
                       b_ref)
    out_ref[0] = y


def _sage_final(sl, sh, cnt, hl, hh, wl, wr, b, slot, carry=None):
    in_specs = list(_DENSE_IN_SPECS)
    args = [sl, sh, cnt, hl, hh, wl, wr, b.reshape(1, DH)]
    aliases = {}
    if carry is not None:
        in_specs.append(pl.BlockSpec(memory_space=pl.ANY))
        args.append(carry)
        aliases = {8: 0}
    return pl.pallas_call(
        functools.partial(_sage_final_body, slot),
        grid=(N // BS,),
        in_specs=in_specs,
        out_specs=pl.BlockSpec((1, BS, DH), lambda i, s=slot: (s, i, 0)),
        out_shape=jax.ShapeDtypeStruct((2, N, DH), jnp.float32),
        input_output_aliases=aliases,
    )(*args)


def _segsum_kernel_body(hlo, hhi, sd_h, zeros_h, out_lo, out_hi,
                        sdidx, slots, acc, semA, semB):
    c = lax.axis_index("c")
    s = lax.axis_index("s")

    def run(table, out):
        zbase = s * ZPT
        pltpu.sync_copy(zeros_h, slots)
        for off, sz in ZCHUNKS:
            pltpu.sync_copy(slots.at[pl.ds(0, sz)],
                            acc.at[pl.ds(zbase + off, sz)])
        plsc.subcore_barrier()

        def body(ci, carry):
            bid = s * NBODIES + ci
            pltpu.sync_copy(sd_h.at[pl.ds(bid * 16, 16)], sdidx)

            def fire_gather(j):
                return pltpu.async_copy(
                    table.at[sdidx.at[j]],
                    slots.at[pl.ds((j % NRING) * SUB, SUB)], semA)

            def fire_scatter(j):
                return pltpu.async_copy(
                    slots.at[pl.ds((j % NRING) * SUB, SUB)],
                    acc.at[sdidx.at[BODY_IDX_ROWS + j]], semB, add=True)

            ga = [fire_gather(j) for j in range(NRING)]
            sc = [None] * BODY_IDX_ROWS
            for j in range(NRING):
                ga[j].wait()
                sc[j] = fire_scatter(j)
            for j in range(NRING, BODY_IDX_ROWS):
                sc[j - NRING].wait()
                ga.append(fire_gather(j))
            for j in range(NRING, BODY_IDX_ROWS):
                ga[j].wait()
                sc[j] = fire_scatter(j)
            for j in range(BODY_IDX_ROWS - NRING, BODY_IDX_ROWS):
                sc[j].wait()
            return carry

        lax.fori_loop(0, NBODIES, body, 0)
        plsc.subcore_barrier()
        for off, sz in ZCHUNKS:
            r0 = zbase + off
            pltpu.sync_copy(acc.at[pl.ds(r0, sz)], slots.at[pl.ds(0, sz)])
            pltpu.sync_copy(slots.at[pl.ds(0, sz)], out.at[pl.ds(r0, sz)])

    @pl.when(c == 0)
    def _():
        run(hlo, out_lo)

    @pl.when(c == 1)
    def _():
        run(hhi, out_hi)


def _segsum(hlo, hhi, sd2d):
    zeros = jnp.zeros((ZBUF, DHALF), jnp.float32)
    mesh = plsc.VectorSubcoreMesh(core_axis_name="c", subcore_axis_name="s")
    f = pl.kernel(
        _segsum_kernel_body,
        compiler_params=pltpu.CompilerParams(use_tc_tiling_on_sc=False),
        out_type=[jax.ShapeDtypeStruct((N_PAD, DHALF), jnp.float32),
                  jax.ShapeDtypeStruct((N_PAD, DHALF), jnp.float32)],
        mesh=mesh,
        scratch_types=[
            pltpu.VMEM((2 * BODY_IDX_ROWS, SUB), jnp.int32),
            pltpu.VMEM((NRING * SUB, DHALF), jnp.float32),
            pltpu.VMEM_SHARED((ACC_ROWS, DHALF), jnp.float32),
            pltpu.SemaphoreType.DMA,
            pltpu.SemaphoreType.DMA,
        ],
    )
    return f(hlo, hhi, sd2d, zeros)


def _counts_kernel_body(sdA_h, sdB_h, zeros_h, ones_h, outA, outB,
                        didx, ones_v, zbuf, acc):
    c = lax.axis_index("c")
    s = lax.axis_index("s")

    def run(sd_h, out):
        zbase = s * ZPT
        pltpu.sync_copy(zeros_h, zbuf)
        for off, sz in ZCHUNKS:
            pltpu.sync_copy(zbuf.at[pl.ds(0, sz)],
                            acc.at[pl.ds(zbase + off, sz)])
        pltpu.sync_copy(ones_h, ones_v)
        plsc.subcore_barrier()

        def chunk(ci, carry):
            bid = s * NBODIES + ci
            pltpu.sync_copy(sd_h.at[pl.ds(bid * 16 + BODY_IDX_ROWS,
                                          BODY_IDX_ROWS)], didx)
            for j in range(BODY_IDX_ROWS):
                pltpu.sync_copy(ones_v, acc.at[didx.at[j]], add=True)
            return carry

        lax.fori_loop(0, NBODIES, chunk, 0)
        plsc.subcore_barrier()
        for off, sz in ZCHUNKS:
            r0 = zbase + off
            pltpu.sync_copy(acc.at[pl.ds(r0, sz)], zbuf.at[pl.ds(0, sz)])
            pltpu.sync_copy(zbuf.at[pl.ds(0, sz)], out.at[pl.ds(r0, sz)])

    @pl.when(c == 0)
    def _():
        run(sdA_h, outA)

    @pl.when(c == 1)
    def _():
        run(sdB_h, outB)


def _counts(sdA2d, sdB2d):
    zeros = jnp.zeros((ZBUF, CW), jnp.float32)
    ones = jnp.ones((SUB, CW), jnp.float32)
    mesh = plsc.VectorSubcoreMesh(core_axis_name="c", subcore_axis_name="s")
    f = pl.kernel(
        _counts_kernel_body,
        compiler_params=pltpu.CompilerParams(use_tc_tiling_on_sc=False),
        out_type=[jax.ShapeDtypeStruct((N_PAD, CW), jnp.float32),
                  jax.ShapeDtypeStruct((N_PAD, CW), jnp.float32)],
        mesh=mesh,
        scratch_types=[
            pltpu.VMEM((BODY_IDX_ROWS, SUB), jnp.int32),
            pltpu.VMEM((SUB, CW), jnp.float32),
            pltpu.VMEM((ZBUF, CW), jnp.float32),
            pltpu.VMEM_SHARED((ACC_ROWS, CW), jnp.float32),
        ],
    )
    return f(sdA2d, sdB2d, zeros, ones)


def _pad_edges(ei):
    src = ei[0].astype(jnp.int32)
    dst = ei[1].astype(jnp.int32)
    npad = E_PAD - E
    src_p = jnp.concatenate([src, jnp.zeros((npad,), jnp.int32)])
    dst_p = jnp.concatenate([dst, jnp.full((npad,), N, jnp.int32)])
    src3 = src_p.reshape(NBODY_TOTAL, BODY_IDX_ROWS, SUB)
    dst3 = dst_p.reshape(NBODY_TOTAL, BODY_IDX_ROWS, SUB)
    return jnp.concatenate([src3, dst3], axis=1).reshape(-1, SUB)


def kernel(x_user, x_item, edge_index_user_to_item, edge_index_item_rev_user,
           Wp_user, bp_user, Wp_item, bp_item, ln_g, ln_b,
           Wl_u2i_0, Wr_u2i_0, b_u2i_0, Wl_i2u_0, Wr_i2u_0, b_i2u_0,
           Wl_u2i_1, Wr_u2i_1, b_u2i_1, Wl_i2u_1, Wr_i2u_1, b_i2u_1):
    sd_u2i = _pad_edges(edge_index_user_to_item)
    sd_i2u = _pad_edges(edge_index_item_rev_user)

    hu_lo, hu_hi = _proj_ln(x_user, Wp_user, bp_user, ln_g, ln_b)
    hi_lo, hi_hi = _proj_ln(x_item, Wp_item, bp_item, ln_g, ln_b)

    cnt_i, cnt_u = _counts(sd_u2i, sd_i2u)

    si_lo, si_hi = _segsum(hu_lo, hu_hi, sd_u2i)
    su_lo, su_hi = _segsum(hi_lo, hi_hi, sd_i2u)

    hi1_lo, hi1_hi = _sage_mid(si_lo, si_hi, cnt_i, hi_lo, hi_hi,
                               Wl_u2i_0, Wr_u2i_0, b_u2i_0)
    hu1_lo, hu1_hi = _sage_mid(su_lo, su_hi, cnt_u, hu_lo, hu_hi,
                               Wl_i2u_0, Wr_i2u_0, b_i2u_0)

    si1_lo, si1_hi = _segsum(hu1_lo, hu1_hi, sd_u2i)
    su1_lo, su1_hi = _segsum(hi1_lo, hi1_hi, sd_i2u)

    out = _sage_final(si1_lo, si1_hi, cnt_i, hi1_lo, hi1_hi,
                      Wl_u2i_1, Wr_u2i_1, b_u2i_1, slot=1)
    out = _sage_final(su1_lo, su1_hi, cnt_u, hu1_lo, hu1_hi,
                      Wl_i2u_1, Wr_i2u_1, b_i2u_1, slot=0, carry=out)
    return out


# speedup vs baseline: 6.5345x; 1.0066x over previous
import functools

import jax
import jax.numpy as jnp
from jax import lax
from jax.experimental import pallas as pl
from jax.experimental.pallas import tpu as pltpu
from jax.experimental.pallas import tpu_sc as plsc

N = 50000
D_IN = 128
DH = 64
DHALF = 32
E = 800000

NUM_SUBCORES = 16
SUB = 128
NRING = 6
BODY_IDX_ROWS = 8
NBODIES = 50
EPT = BODY_IDX_ROWS * SUB * NBODIES
E_PAD = EPT * NUM_SUBCORES
NBODY_TOTAL = E_PAD // (BODY_IDX_ROWS * SUB)

N_PAD = 50048
ACC_ROWS = N_PAD
ZPT = N_PAD // NUM_SUBCORES
ZCHUNKS = ((0, 768), (768, 768), (1536, 768), (2304, 768), (3072, 56))
ZBUF = 768

CW = 8
BS = 1000


def _proj_ln_body(x_ref, w_ref, b_ref, g_ref, b2_ref, lo_ref, hi_ref):
    y = jnp.dot(x_ref[...], w_ref[...], preferred_element_type=jnp.float32)
    y = y + b_ref[...]
    m = jnp.mean(y, axis=-1, keepdims=True)
    v = jnp.mean((y - m) ** 2, axis=-1, keepdims=True)
    y = (y - m) * lax.rsqrt(v + 1e-5) * g_ref[...] + b2_ref[...]
    lo_ref[...] = y[:, :DHALF]
    hi_ref[...] = y[:, DHALF:]


def _proj_ln(x, w, b, g, b2):
    return pl.pallas_call(
        _proj_ln_body,
        grid=(N // BS,),
        in_specs=[
            pl.BlockSpec((BS, D_IN), lambda i: (i, 0)),
            pl.BlockSpec((D_IN, DH), lambda i: (0, 0)),
            pl.BlockSpec((1, DH), lambda i: (0, 0)),
            pl.BlockSpec((1, DH), lambda i: (0, 0)),
            pl.BlockSpec((1, DH), lambda i: (0, 0)),
        ],
        out_specs=[
            pl.BlockSpec((BS, DHALF), lambda i: (i, 0)),
            pl.BlockSpec((BS, DHALF), lambda i: (i, 0)),
        ],
        out_shape=[
            jax.ShapeDtypeStruct((N, DHALF), jnp.float32),
            jax.ShapeDtypeStruct((N, DHALF), jnp.float32),
        ],
    )(x, w, b.reshape(1, DH), g.reshape(1, DH), b2.reshape(1, DH))


def _dense_compute(sl_ref, sh_ref, c_ref, hl_ref, hh_ref, wl_ref, wr_ref,
                   b_ref):
    cnt = c_ref[...][:, 0:1]
    r = 1.0 / jnp.maximum(cnt, 1.0)
    ml = sl_ref[...] * r
    mh = sh_ref[...] * r
    y = jnp.dot(ml, wl_ref[:DHALF, :], preferred_element_type=jnp.float32)
    y = y + jnp.dot(mh, wl_ref[DHALF:, :], preferred_element_type=jnp.float32)
    y = y + jnp.dot(hl_ref[...], wr_ref[:DHALF, :],
                    preferred_element_type=jnp.float32)
    y = y + jnp.dot(hh_ref[...], wr_ref[DHALF:, :],
                    preferred_element_type=jnp.float32)
    return y + b_ref[...]


def _sage_mid_body(sl_ref, sh_ref, c_ref, hl_ref, hh_ref, wl_ref, wr_ref,
                   b_ref, lo_ref, hi_ref):
    y = _dense_compute(sl_ref, sh_ref, c_ref, hl_ref, hh_ref, wl_ref, wr_ref,
                       b_ref)
    y = jnp.maximum(y, 0.0)
    lo_ref[...] = y[:, :DHALF]
    hi_ref[...] = y[:, DHALF:]


_DENSE_IN_SPECS = [
    pl.BlockSpec((BS, DHALF), lambda i: (i, 0)),
    pl.BlockSpec((BS, DHALF), lambda i: (i, 0)),
    pl.BlockSpec((BS, CW), lambda i: (i, 0)),
    pl.BlockSpec((BS, DHALF), lambda i: (i, 0)),
    pl.BlockSpec((BS, DHALF), lambda i: (i, 0)),
    pl.BlockSpec((DH, DH), lambda i: (0, 0)),
    pl.BlockSpec((DH, DH), lambda i: (0, 0)),
    pl.BlockSpec((1, DH), lambda i: (0, 0)),
]


def _sage_mid(sl, sh, cnt, hl, hh, wl, wr, b):
    return pl.pallas_call(
        _sage_mid_body,
        grid=(N // BS,),
        in_specs=_DENSE_IN_SPECS,
        out_specs=[pl.BlockSpec((BS, DHALF), lambda i: (i, 0)),
                   pl.BlockSpec((BS, DHALF), lambda i: (i, 0))],
        out_shape=[jax.ShapeDtypeStruct((N, DHALF), jnp.float32),
                   jax.ShapeDtypeStruct((N, DHALF), jnp.float32)],
    )(sl, sh, cnt, hl, hh, wl, wr, b.reshape(1, DH))


def _sage_final_body(slot, sl_ref, sh_ref, c_ref, hl_ref, hh_ref, wl_ref,
                     wr_ref, b_ref, *refs):
    out_ref = refs[-1]
    y = _dense_compute(sl_ref, sh_ref, c_ref, hl_ref, hh_ref, wl_ref, wr_ref,
                       b_ref)
    out_ref[0] = y


def _sage_final(sl, sh, cnt, hl, hh, wl, wr, b, slot, carry=None):
    in_specs = list(_DENSE_IN_SPECS)
    args = [sl, sh, cnt, hl, hh, wl, wr, b.reshape(1, DH)]
    aliases = {}
    if carry is not None:
        in_specs.append(pl.BlockSpec(memory_space=pl.ANY))
        args.append(carry)
        aliases = {8: 0}
    return pl.pallas_call(
        functools.partial(_sage_final_body, slot),
        grid=(N // BS,),
        in_specs=in_specs,
        out_specs=pl.BlockSpec((1, BS, DH), lambda i, s=slot: (s, i, 0)),
        out_shape=jax.ShapeDtypeStruct((2, N, DH), jnp.float32),
        input_output_aliases=aliases,
    )(*args)


def _segsum_kernel_body(hlo, hhi, src_h, dst_h, zeros_h, out_lo, out_hi,
                        sidx, didx, slots, acc, semA, semB):
    c = lax.axis_index("c")
    s = lax.axis_index("s")

    def run(table, out):
        zbase = s * ZPT
        pltpu.sync_copy(zeros_h, slots)
        for off, sz in ZCHUNKS:
            pltpu.sync_copy(slots.at[pl.ds(0, sz)],
                            acc.at[pl.ds(zbase + off, sz)])
        plsc.subcore_barrier()

        def body(ci, carry):
            bid = s * NBODIES + ci
            pltpu.sync_copy(src_h.at[pl.ds(bid * BODY_IDX_ROWS,
                                           BODY_IDX_ROWS)], sidx)
            pltpu.sync_copy(dst_h.at[pl.ds(bid * BODY_IDX_ROWS,
                                           BODY_IDX_ROWS)], didx)

            def fire_gather(j):
                return pltpu.async_copy(
                    table.at[sidx.at[j]],
                    slots.at[pl.ds((j % NRING) * SUB, SUB)], semA)

            def fire_scatter(j):
                return pltpu.async_copy(
                    slots.at[pl.ds((j % NRING) * SUB, SUB)],
                    acc.at[didx.at[j]], semB, add=True)

            ga = [fire_gather(j) for j in range(NRING)]
            sc = [None] * BODY_IDX_ROWS
            for j in range(NRING):
                ga[j].wait()
                sc[j] = fire_scatter(j)
            for j in range(NRING, BODY_IDX_ROWS):
                sc[j - NRING].wait()
                ga.append(fire_gather(j))
            for j in range(NRING, BODY_IDX_ROWS):
                ga[j].wait()
                sc[j] = fire_scatter(j)
            for j in range(BODY_IDX_ROWS - NRING, BODY_IDX_ROWS):
                sc[j].wait()
            return carry

        lax.fori_loop(0, NBODIES, body, 0)
        plsc.subcore_barrier()
        for off, sz in ZCHUNKS:
            r0 = zbase + off
            pltpu.sync_copy(acc.at[pl.ds(r0, sz)], slots.at[pl.ds(0, sz)])
            pltpu.sync_copy(slots.at[pl.ds(0, sz)], out.at[pl.ds(r0, sz)])

    @pl.when(c == 0)
    def _():
        run(hlo, out_lo)

    @pl.when(c == 1)
    def _():
        run(hhi, out_hi)


def _segsum(hlo, hhi, src2d, dst2d):
    zeros = jnp.zeros((ZBUF, DHALF), jnp.float32)
    mesh = plsc.VectorSubcoreMesh(core_axis_name="c", subcore_axis_name="s")
    f = pl.kernel(
        _segsum_kernel_body,
        compiler_params=pltpu.CompilerParams(use_tc_tiling_on_sc=False),
        out_type=[jax.ShapeDtypeStruct((N_PAD, DHALF), jnp.float32),
                  jax.ShapeDtypeStruct((N_PAD, DHALF), jnp.float32)],
        mesh=mesh,
        scratch_types=[
            pltpu.VMEM((BODY_IDX_ROWS, SUB), jnp.int32),
            pltpu.VMEM((BODY_IDX_ROWS, SUB), jnp.int32),
            pltpu.VMEM((NRING * SUB, DHALF), jnp.float32),
            pltpu.VMEM_SHARED((ACC_ROWS, DHALF), jnp.float32),
            pltpu.SemaphoreType.DMA,
            pltpu.SemaphoreType.DMA,
        ],
    )
    return f(hlo, hhi, src2d, dst2d, zeros)


def _counts_kernel_body(dstA_h, dstB_h, zeros_h, ones_h, outA, outB,
                        didx, ones_v, zbuf, acc):
    c = lax.axis_index("c")
    s = lax.axis_index("s")

    def run(dst_h, out):
        zbase = s * ZPT
        pltpu.sync_copy(zeros_h, zbuf)
        for off, sz in ZCHUNKS:
            pltpu.sync_copy(zbuf.at[pl.ds(0, sz)],
                            acc.at[pl.ds(zbase + off, sz)])
        pltpu.sync_copy(ones_h, ones_v)
        plsc.subcore_barrier()

        def chunk(ci, carry):
            bid = s * NBODIES + ci
            pltpu.sync_copy(dst_h.at[pl.ds(bid * BODY_IDX_ROWS,
                                           BODY_IDX_ROWS)], didx)
            for j in range(BODY_IDX_ROWS):
                pltpu.sync_copy(ones_v, acc.at[didx.at[j]], add=True)
            return carry

        lax.fori_loop(0, NBODIES, chunk, 0)
        plsc.subcore_barrier()
        for off, sz in ZCHUNKS:
            r0 = zbase + off
            pltpu.sync_copy(acc.at[pl.ds(r0, sz)], zbuf.at[pl.ds(0, sz)])
            pltpu.sync_copy(zbuf.at[pl.ds(0, sz)], out.at[pl.ds(r0, sz)])

    @pl.when(c == 0)
    def _():
        run(dstA_h, outA)

    @pl.when(c == 1)
    def _():
        run(dstB_h, outB)


def _counts(dstA2d, dstB2d):
    zeros = jnp.zeros((ZBUF, CW), jnp.float32)
    ones = jnp.ones((SUB, CW), jnp.float32)
    mesh = plsc.VectorSubcoreMesh(core_axis_name="c", subcore_axis_name="s")
    f = pl.kernel(
        _counts_kernel_body,
        compiler_params=pltpu.CompilerParams(use_tc_tiling_on_sc=False),
        out_type=[jax.ShapeDtypeStruct((N_PAD, CW), jnp.float32),
                  jax.ShapeDtypeStruct((N_PAD, CW), jnp.float32)],
        mesh=mesh,
        scratch_types=[
            pltpu.VMEM((BODY_IDX_ROWS, SUB), jnp.int32),
            pltpu.VMEM((SUB, CW), jnp.float32),
            pltpu.VMEM((ZBUF, CW), jnp.float32),
            pltpu.VMEM_SHARED((ACC_ROWS, CW), jnp.float32),
        ],
    )
    return f(dstA2d, dstB2d, zeros, ones)


def _pad_edges(ei):
    src = ei[0].astype(jnp.int32)
    dst = ei[1].astype(jnp.int32)
    npad = E_PAD - E
    src_p = jnp.concatenate([src, jnp.zeros((npad,), jnp.int32)])
    dst_p = jnp.concatenate([dst, jnp.full((npad,), N, jnp.int32)])
    return src_p.reshape(-1, SUB), dst_p.reshape(-1, SUB)


def kernel(x_user, x_item, edge_index_user_to_item, edge_index_item_rev_user,
           Wp_user, bp_user, Wp_item, bp_item, ln_g, ln_b,
           Wl_u2i_0, Wr_u2i_0, b_u2i_0, Wl_i2u_0, Wr_i2u_0, b_i2u_0,
           Wl_u2i_1, Wr_u2i_1, b_u2i_1, Wl_i2u_1, Wr_i2u_1, b_i2u_1):
    src_u2i, dst_u2i = _pad_edges(edge_index_user_to_item)
    src_i2u, dst_i2u = _pad_edges(edge_index_item_rev_user)

    hu_lo, hu_hi = _proj_ln(x_user, Wp_user, bp_user, ln_g, ln_b)
    hi_lo, hi_hi = _proj_ln(x_item, Wp_item, bp_item, ln_g, ln_b)

    cnt_i, cnt_u = _counts(dst_u2i, dst_i2u)

    si_lo, si_hi = _segsum(hu_lo, hu_hi, src_u2i, dst_u2i)
    su_lo, su_hi = _segsum(hi_lo, hi_hi, src_i2u, dst_i2u)

    hi1_lo, hi1_hi = _sage_mid(si_lo, si_hi, cnt_i, hi_lo, hi_hi,
                               Wl_u2i_0, Wr_u2i_0, b_u2i_0)
    hu1_lo, hu1_hi = _sage_mid(su_lo, su_hi, cnt_u, hu_lo, hu_hi,
                               Wl_i2u_0, Wr_i2u_0, b_i2u_0)

    si1_lo, si1_hi = _segsum(hu1_lo, hu1_hi, src_u2i, dst_u2i)
    su1_lo, su1_hi = _segsum(hi1_lo, hi1_hi, src_i2u, dst_i2u)

    out = _sage_final(si1_lo, si1_hi, cnt_i, hi1_lo, hi1_hi,
                      Wl_u2i_1, Wr_u2i_1, b_u2i_1, slot=1)
    out = _sage_final(su1_lo, su1_hi, cnt_u, hu1_lo, hu1_hi,
                      Wl_i2u_1, Wr_i2u_1, b_i2u_1, slot=0, carry=out)
    return out


# speedup vs baseline: 6.9039x; 1.0565x over previous
import functools

import jax
import jax.numpy as jnp
from jax import lax
from jax.experimental import pallas as pl
from jax.experimental.pallas import tpu as pltpu
from jax.experimental.pallas import tpu_sc as plsc

N = 50000
D_IN = 128
DH = 64
DHALF = 32
E = 800000

NUM_SUBCORES = 16
SUB = 128
NRING = 6
BODY_IDX_ROWS = 8
NBODIES = 50
EPT = BODY_IDX_ROWS * SUB * NBODIES
E_PAD = EPT * NUM_SUBCORES
NBODY_TOTAL = E_PAD // (BODY_IDX_ROWS * SUB)

N_PAD = 50048
ACC_ROWS = N_PAD
ZPT = N_PAD // NUM_SUBCORES
ZCHUNKS = tuple((i * 256, 256) for i in range(12)) + ((3072, 56),)
ZBUF = 256

CW = 16
BS = 1000


def _proj_ln_body(x_ref, w_ref, b_ref, g_ref, b2_ref, lo_ref, hi_ref):
    y = jnp.dot(x_ref[...], w_ref[...], preferred_element_type=jnp.float32)
    y = y + b_ref[...]
    m = jnp.mean(y, axis=-1, keepdims=True)
    v = jnp.mean((y - m) ** 2, axis=-1, keepdims=True)
    y = (y - m) * lax.rsqrt(v + 1e-5) * g_ref[...] + b2_ref[...]
    lo_ref[...] = y[:, :DHALF]
    hi_ref[...] = y[:, DHALF:]


def _proj_ln(x, w, b, g, b2):
    return pl.pallas_call(
        _proj_ln_body,
        grid=(N // BS,),
        in_specs=[
            pl.BlockSpec((BS, D_IN), lambda i: (i, 0)),
            pl.BlockSpec((D_IN, DH), lambda i: (0, 0)),
            pl.BlockSpec((1, DH), lambda i: (0, 0)),
            pl.BlockSpec((1, DH), lambda i: (0, 0)),
            pl.BlockSpec((1, DH), lambda i: (0, 0)),
        ],
        out_specs=[
            pl.BlockSpec((BS, DHALF), lambda i: (i, 0)),
            pl.BlockSpec((BS, DHALF), lambda i: (i, 0)),
        ],
        out_shape=[
            jax.ShapeDtypeStruct((N, DHALF), jnp.float32),
            jax.ShapeDtypeStruct((N, DHALF), jnp.float32),
        ],
    )(x, w, b.reshape(1, DH), g.reshape(1, DH), b2.reshape(1, DH))


def _dense_compute(sl_ref, sh_ref, c_ref, hl_ref, hh_ref, wl_ref, wr_ref,
                   b_ref):
    cnt = c_ref[...][:, 0:1]
    r = 1.0 / jnp.maximum(cnt, 1.0)
    ml = sl_ref[...] * r
    mh = sh_ref[...] * r
    y = jnp.dot(ml, wl_ref[:DHALF, :], preferred_element_type=jnp.float32)
    y = y + jnp.dot(mh, wl_ref[DHALF:, :], preferred_element_type=jnp.float32)
    y = y + jnp.dot(hl_ref[...], wr_ref[:DHALF, :],
                    preferred_element_type=jnp.float32)
    y = y + jnp.dot(hh_ref[...], wr_ref[DHALF:, :],
                    preferred_element_type=jnp.float32)
    return y + b_ref[...]


def _sage_mid_body(sl_ref, sh_ref, c_ref, hl_ref, hh_ref, wl_ref, wr_ref,
                   b_ref, lo_ref, hi_ref):
    y = _dense_compute(sl_ref, sh_ref, c_ref, hl_ref, hh_ref, wl_ref, wr_ref,
                       b_ref)
    y = jnp.maximum(y, 0.0)
    lo_ref[...] = y[:, :DHALF]
    hi_ref[...] = y[:, DHALF:]


_DENSE_IN_SPECS = [
    pl.BlockSpec((BS, DHALF), lambda i: (i, 0)),
    pl.BlockSpec((BS, DHALF), lambda i: (i, 0)),
    pl.BlockSpec((BS, CW), lambda i: (i, 0)),
    pl.BlockSpec((BS, DHALF), lambda i: (i, 0)),
    pl.BlockSpec((BS, DHALF), lambda i: (i, 0)),
    pl.BlockSpec((DH, DH), lambda i: (0, 0)),
    pl.BlockSpec((DH, DH), lambda i: (0, 0)),
    pl.BlockSpec((1, DH), lambda i: (0, 0)),
]


def _sage_mid(sl, sh, cnt, hl, hh, wl, wr, b):
    return pl.pallas_call(
        _sage_mid_body,
        grid=(N // BS,),
        in_specs=_DENSE_IN_SPECS,
        out_specs=[pl.BlockSpec((BS, DHALF), lambda i: (i, 0)),
                   pl.BlockSpec((BS, DHALF), lambda i: (i, 0))],
        out_shape=[jax.ShapeDtypeStruct((N, DHALF), jnp.float32),
                   jax.ShapeDtypeStruct((N, DHALF), jnp.float32)],
    )(sl, sh, cnt, hl, hh, wl, wr, b.reshape(1, DH))


def _sage_final_body(sl_ref, sh_ref, c_ref, hl_ref, hh_ref, wl_ref,
                     wr_ref, b_ref, out_ref):
    out_ref[...] = _dense_compute(sl_ref, sh_ref, c_ref, hl_ref, hh_ref,
                                  wl_ref, wr_ref, b_ref)


def _sage_final(sl, sh, cnt, hl, hh, wl, wr, b):
    return pl.pallas_call(
        _sage_final_body,
        grid=(N // BS,),
        in_specs=_DENSE_IN_SPECS,
        out_specs=pl.BlockSpec((BS, DH), lambda i: (i, 0)),
        out_shape=jax.ShapeDtypeStruct((N, DH), jnp.float32),
    )(sl, sh, cnt, hl, hh, wl, wr, b.reshape(1, DH))


def _segsum_kernel_body(hlo, hhi, src_h, dst_h, zeros_h, out_lo, out_hi,
                        sidx, didx, slots, acc, semA, semB):
    c = lax.axis_index("c")
    s = lax.axis_index("s")

    def run(table, out):
        zbase = s * ZPT
        pltpu.sync_copy(zeros_h, slots.at[pl.ds(0, ZBUF)])
        for off, sz in ZCHUNKS:
            pltpu.sync_copy(slots.at[pl.ds(0, sz)],
                            acc.at[pl.ds(zbase + off, sz)])
        plsc.subcore_barrier()

        def body(ci, carry):
            bid = s * NBODIES + ci
            pltpu.sync_copy(src_h.at[pl.ds(bid * BODY_IDX_ROWS,
                                           BODY_IDX_ROWS)], sidx)
            pltpu.sync_copy(dst_h.at[pl.ds(bid * BODY_IDX_ROWS,
                                           BODY_IDX_ROWS)], didx)

            def fire_gather(j):
                return pltpu.async_copy(
                    table.at[sidx.at[j]],
                    slots.at[pl.ds((j % NRING) * SUB, SUB)], semA)

            def fire_scatter(j):
                return pltpu.async_copy(
                    slots.at[pl.ds((j % NRING) * SUB, SUB)],
                    acc.at[didx.at[j]], semB, add=True)

            ga = [fire_gather(j) for j in range(NRING)]
            sc = [None] * BODY_IDX_ROWS
            for j in range(NRING):
                ga[j].wait()
                sc[j] = fire_scatter(j)
            for j in range(NRING, BODY_IDX_ROWS):
                sc[j - NRING].wait()
                ga.append(fire_gather(j))
            for j in range(NRING, BODY_IDX_ROWS):
                ga[j].wait()
                sc[j] = fire_scatter(j)
            for j in range(BODY_IDX_ROWS - NRING, BODY_IDX_ROWS):
                sc[j].wait()
            return carry

        lax.fori_loop(0, NBODIES, body, 0)
        plsc.subcore_barrier()
        for off, sz in ZCHUNKS:
            r0 = zbase + off
            pltpu.sync_copy(acc.at[pl.ds(r0, sz)], slots.at[pl.ds(0, sz)])
            pltpu.sync_copy(slots.at[pl.ds(0, sz)], out.at[pl.ds(r0, sz)])

    @pl.when(c == 0)
    def _():
        run(hlo, out_lo)

    @pl.when(c == 1)
    def _():
        run(hhi, out_hi)


def _segsum(hlo, hhi, src2d, dst2d):
    zeros = jnp.zeros((ZBUF, DHALF), jnp.float32)
    mesh = plsc.VectorSubcoreMesh(core_axis_name="c", subcore_axis_name="s")
    f = pl.kernel(
        _segsum_kernel_body,
        compiler_params=pltpu.CompilerParams(use_tc_tiling_on_sc=False),
        out_type=[jax.ShapeDtypeStruct((N_PAD, DHALF), jnp.float32),
                  jax.ShapeDtypeStruct((N_PAD, DHALF), jnp.float32)],
        mesh=mesh,
        scratch_types=[
            pltpu.VMEM((BODY_IDX_ROWS, SUB), jnp.int32),
            pltpu.VMEM((BODY_IDX_ROWS, SUB), jnp.int32),
            pltpu.VMEM((NRING * SUB, DHALF), jnp.float32),
            pltpu.VMEM_SHARED((ACC_ROWS, DHALF), jnp.float32),
            pltpu.SemaphoreType.DMA,
            pltpu.SemaphoreType.DMA,
        ],
    )
    return f(hlo, hhi, src2d, dst2d, zeros)


def _counts_kernel_body(dstA_h, dstB_h, zeros_h, ones_h, outA, outB,
                        didx, ones_v, zbuf, acc):
    c = lax.axis_index("c")
    s = lax.axis_index("s")

    def run(dst_h, out):
        zbase = s * ZPT
        pltpu.sync_copy(zeros_h, zbuf)
        for off, sz in ZCHUNKS:
            pltpu.sync_copy(zbuf.at[pl.ds(0, sz)],
                            acc.at[pl.ds(zbase + off, sz)])
        pltpu.sync_copy(ones_h, ones_v)
        plsc.subcore_barrier()

        def chunk(ci, carry):
            bid = s * NBODIES + ci
            pltpu.sync_copy(dst_h.at[pl.ds(bid * BODY_IDX_ROWS,
                                           BODY_IDX_ROWS)], didx)
            for j in range(BODY_IDX_ROWS):
                pltpu.sync_copy(ones_v, acc.at[didx.at[j]], add=True)
            return carry

        lax.fori_loop(0, NBODIES, chunk, 0)
        plsc.subcore_barrier()
        for off, sz in ZCHUNKS:
            r0 = zbase + off
            pltpu.sync_copy(acc.at[pl.ds(r0, sz)], zbuf.at[pl.ds(0, sz)])
            pltpu.sync_copy(zbuf.at[pl.ds(0, sz)], out.at[pl.ds(r0, sz)])

    @pl.when(c == 0)
    def _():
        run(dstA_h, outA)

    @pl.when(c == 1)
    def _():
        run(dstB_h, outB)


def _counts(dstA2d, dstB2d):
    zeros = jnp.zeros((ZBUF, CW), jnp.float32)
    ones = jnp.ones((SUB, CW), jnp.float32)
    mesh = plsc.VectorSubcoreMesh(core_axis_name="c", subcore_axis_name="s")
    f = pl.kernel(
        _counts_kernel_body,
        compiler_params=pltpu.CompilerParams(use_tc_tiling_on_sc=False),
        out_type=[jax.ShapeDtypeStruct((N_PAD, CW), jnp.float32),
                  jax.ShapeDtypeStruct((N_PAD, CW), jnp.float32)],
        mesh=mesh,
        scratch_types=[
            pltpu.VMEM((BODY_IDX_ROWS, SUB), jnp.int32),
            pltpu.VMEM((SUB, CW), jnp.float32),
            pltpu.VMEM((ZBUF, CW), jnp.float32),
            pltpu.VMEM_SHARED((ACC_ROWS, CW), jnp.float32),
        ],
    )
    return f(dstA2d, dstB2d, zeros, ones)


def _pad_edges(ei):
    src = ei[0].astype(jnp.int32)
    dst = ei[1].astype(jnp.int32)
    npad = E_PAD - E
    src_p = jnp.concatenate([src, jnp.zeros((npad,), jnp.int32)])
    dst_p = jnp.concatenate([dst, jnp.full((npad,), N, jnp.int32)])
    return src_p.reshape(-1, SUB), dst_p.reshape(-1, SUB)


def kernel(x_user, x_item, edge_index_user_to_item, edge_index_item_rev_user,
           Wp_user, bp_user, Wp_item, bp_item, ln_g, ln_b,
           Wl_u2i_0, Wr_u2i_0, b_u2i_0, Wl_i2u_0, Wr_i2u_0, b_i2u_0,
           Wl_u2i_1, Wr_u2i_1, b_u2i_1, Wl_i2u_1, Wr_i2u_1, b_i2u_1):
    src_u2i, dst_u2i = _pad_edges(edge_index_user_to_item)
    src_i2u, dst_i2u = _pad_edges(edge_index_item_rev_user)

    hu_lo, hu_hi = _proj_ln(x_user, Wp_user, bp_user, ln_g, ln_b)
    hi_lo, hi_hi = _proj_ln(x_item, Wp_item, bp_item, ln_g, ln_b)

    cnt_i, cnt_u = _counts(dst_u2i, dst_i2u)

    si_lo, si_hi = _segsum(hu_lo, hu_hi, src_u2i, dst_u2i)
    su_lo, su_hi = _segsum(hi_lo, hi_hi, src_i2u, dst_i2u)

    hi1_lo, hi1_hi = _sage_mid(si_lo, si_hi, cnt_i, hi_lo, hi_hi,
                               Wl_u2i_0, Wr_u2i_0, b_u2i_0)
    hu1_lo, hu1_hi = _sage_mid(su_lo, su_hi, cnt_u, hu_lo, hu_hi,
                               Wl_i2u_0, Wr_i2u_0, b_i2u_0)

    si1_lo, si1_hi = _segsum(hu1_lo, hu1_hi, src_u2i, dst_u2i)
    su1_lo, su1_hi = _segsum(hi1_lo, hi1_hi, src_i2u, dst_i2u)

    hi2 = _sage_final(si1_lo, si1_hi, cnt_i, hi1_lo, hi1_hi,
                      Wl_u2i_1, Wr_u2i_1, b_u2i_1)
    hu2 = _sage_final(su1_lo, su1_hi, cnt_u, hu1_lo, hu1_hi,
                      Wl_i2u_1, Wr_i2u_1, b_i2u_1)
    return jnp.stack([hu2, hi2], axis=0)
